# baseline (speedup 1.0000x reference)
import jax
import jax.numpy as jnp
from jax.experimental import pallas as pl
from jax.experimental.pallas import tpu as pltpu

N = 8192
KSEL = 30
ROWS = 256
HID = 100
HPAD = 128
DT = 0.5


def _body(ur_ref, sr_ref, ua_ref, sa_ref, p8r_ref, p8c_ref,
          w1u_ref, w1s_ref, b1_ref, w2_ref, b2_ref, w3_ref, b3_ref,
          a0_ref, be_ref, ga_ref,
          cost_ref, u1_ref, s1_ref, al_ref,
          keep_ref, stat_ref):
    ur = ur_ref[...]
    sr = sr_ref[...]
    ua = ua_ref[...]
    sa = sa_ref[...]
    alpha0 = a0_ref[0, 0]
    beta0 = be_ref[0, 0]
    gamma0 = ga_ref[0, 0]

    pre1 = (ur * w1u_ref[...] + sr * w1s_ref[...]) + b1_ref[...]
    h1 = jax.nn.sigmoid(pre1)
    h2 = jax.nn.sigmoid(
        jnp.dot(h1, w2_ref[...], preferred_element_type=jnp.float32)
        + b2_ref[...])
    apre = jnp.sum(h2 * w3_ref[...], axis=1, keepdims=True) + b3_ref[0, 0]
    alphas = jax.nn.sigmoid(apre) * alpha0

    u1 = ur + (alphas - beta0 * ur) * DT
    s1 = sr + (beta0 * ur - gamma0 * sr) * DT
    uv = u1 - ur
    sv = s1 - sr

    sqa = sa * sa + ua * ua
    sqr = sr * sr + ur * ur
    dotv = jnp.dot(p8r_ref[...], p8c_ref[...],
                   preferred_element_type=jnp.float32)
    d2 = (sqr + sqa) - 2.0 * dotv

    gmin = jnp.min(d2, axis=1, keepdims=True)

    kf = jnp.float32(KSEL)

    stat_ref[:, 0:1] = gmin
    stat_ref[:, 1:2] = gmin
    stat_ref[:, 2:3] = jnp.zeros((ROWS, 1), jnp.float32)
    stat_ref[:, 3:4] = jnp.zeros((ROWS, 1), jnp.float32)
    stat_ref[:, 4:5] = jnp.zeros((ROWS, 1), jnp.float32)

    def sel_body(k, carry):
        tcur = stat_ref[:, 0:1]
        done = stat_ref[:, 4:5] > 0.0
        vnext = jnp.min(jnp.where(d2 > tcur, d2, jnp.inf),
                        axis=1, keepdims=True)
        cle = jnp.sum((d2 <= tcur).astype(jnp.float32),
                      axis=1, keepdims=True)
        cross = jnp.logical_not(done) & (cle >= kf)
        stat_ref[:, 1:2] = jnp.where(cross, tcur, stat_ref[:, 1:2])
        stat_ref[:, 2:3] = jnp.where(cross, stat_ref[:, 3:4],
                                     stat_ref[:, 2:3])
        newdone = done | cross
        stat_ref[:, 4:5] = newdone.astype(jnp.float32)
        stat_ref[:, 0:1] = jnp.where(newdone, tcur, vnext)
        stat_ref[:, 3:4] = jnp.where(newdone, stat_ref[:, 3:4], cle)
        return carry

    jax.lax.fori_loop(0, KSEL, sel_body, 0)

    t = stat_ref[:, 1:2]
    clt = stat_ref[:, 2:3]

    tie = d2 == t
    ntie = jnp.sum(tie.astype(jnp.float32), axis=1, keepdims=True)
    excess0 = ntie - (kf - clt)
    keep_ref[...] = tie.astype(jnp.float32)
    stat_ref[:, 5:6] = excess0
    idx = jax.lax.broadcasted_iota(jnp.int32, (ROWS, N), 1)

    def trim_cond(nleft):
        return nleft > 0.0

    def trim_body(nleft):
        excess = stat_ref[:, 5:6]
        km = keep_ref[...] > 0.0
        jmax = jnp.max(jnp.where(km, idx, -1), axis=1, keepdims=True)
        upd = excess > 0.0
        rm = upd & (idx == jmax)
        keep_ref[...] = jnp.where(rm, 0.0, keep_ref[...])
        exnew = jnp.where(upd, excess - 1.0, excess)
        stat_ref[:, 5:6] = exnew
        return jnp.max(exnew)

    jax.lax.while_loop(trim_cond, trim_body, jnp.max(excess0))

    jmin0 = jnp.min(jnp.where(d2 == gmin, idx, N), axis=1, keepdims=True)
    first = (d2 == gmin) & (idx == jmin0)
    included = ((d2 < t) | (keep_ref[...] > 0.0)) & jnp.logical_not(first)

    unv = ua - ur
    snv = sa - sr
    den = jnp.sqrt(unv * unv + snv * snv) * jnp.sqrt(uv * uv + sv * sv)
    num = unv * uv + snv * sv
    den_safe = jnp.where(den == 0.0, 1.0, den)
    cosine = jnp.where(den == 0.0, 1.0, num / den_safe)
    cosmax = jnp.max(jnp.where(included, cosine, -3.0), axis=1, keepdims=True)

    cost_ref[...] = 1.0 - cosmax
    u1_ref[...] = u1
    s1_ref[...] = s1
    al_ref[...] = alphas


def kernel(u0, s0, alpha0, beta0, gamma0, W1, b1, W2, b2, W3, b3):
    f32 = jnp.float32
    u_col = u0.reshape(N, 1)
    s_col = s0.reshape(N, 1)
    u_row = u0.reshape(1, N)
    s_row = s0.reshape(1, N)
    pts8 = jnp.pad(jnp.stack([s0, u0], axis=1), ((0, 0), (0, 6)))
    pts8t = pts8.T
    hp = HPAD - HID
    w1u = jnp.pad(W1[0:1, :], ((0, 0), (0, hp)))
    w1s = jnp.pad(W1[1:2, :], ((0, 0), (0, hp)))
    b1p = jnp.pad(b1.reshape(1, HID), ((0, 0), (0, hp)))
    w2p = jnp.pad(W2, ((0, hp), (0, hp)))
    b2p = jnp.pad(b2.reshape(1, HID), ((0, 0), (0, hp)))
    w3p = jnp.pad(W3.reshape(1, HID), ((0, 0), (0, hp)))
    b3p = b3.reshape(1, 1)
    a0 = alpha0.reshape(1, 1).astype(f32)
    be = beta0.reshape(1, 1).astype(f32)
    ga = gamma0.reshape(1, 1).astype(f32)

    grid = N // ROWS
    row_spec = pl.BlockSpec((ROWS, 1), lambda i: (i, 0))
    full_spec = pl.BlockSpec((1, N), lambda i: (0, 0))

    def fixed(shape):
        return pl.BlockSpec(shape, lambda i: (0, 0))

    cost2, u12, s12, al2 = pl.pallas_call(
        _body,
        grid=(grid,),
        in_specs=[row_spec, row_spec, full_spec, full_spec,
                  pl.BlockSpec((ROWS, 8), lambda i: (i, 0)),
                  pl.BlockSpec((8, N), lambda i: (0, 0)),
                  fixed((1, HPAD)), fixed((1, HPAD)), fixed((1, HPAD)),
                  fixed((HPAD, HPAD)), fixed((1, HPAD)), fixed((1, HPAD)),
                  fixed((1, 1)), fixed((1, 1)), fixed((1, 1)), fixed((1, 1))],
        out_specs=[row_spec, row_spec, row_spec, row_spec],
        out_shape=[jax.ShapeDtypeStruct((N, 1), f32) for _ in range(4)],
        scratch_shapes=[pltpu.VMEM((ROWS, N), f32),
                        pltpu.VMEM((ROWS, 128), f32)],
    )(u_col, s_col, u_row, s_row, pts8, pts8t,
      w1u, w1s, b1p, w2p, b2p, w3p, b3p, a0, be, ga)

    cost = cost2.reshape(N)
    u1 = u12.reshape(N)
    s1 = s12.reshape(N)
    alphas = al2.reshape(N)
    beta = jnp.broadcast_to(beta0, u0.shape)
    gamma = jnp.broadcast_to(gamma0, u0.shape)
    return (cost, u1, s1, alphas, beta, gamma)


# speedup vs baseline: 2.2875x; 2.2875x over previous
import jax
import jax.numpy as jnp
from jax.experimental import pallas as pl
from jax.experimental.pallas import tpu as pltpu

N = 8192
KSEL = 30
ROWS = 256
HID = 100
HPAD = 128
DT = 0.5


def _body(ur_ref, sr_ref, ua_ref, sa_ref, p8r_ref, p8c_ref,
          w1u_ref, w1s_ref, b1_ref, w2_ref, b2_ref, w3_ref, b3_ref,
          a0_ref, be_ref, ga_ref,
          cost_ref, u1_ref, s1_ref, al_ref,
          keep_ref, stat_ref):
    ur = ur_ref[...]
    sr = sr_ref[...]
    ua = ua_ref[...]
    sa = sa_ref[...]
    alpha0 = a0_ref[0, 0]
    beta0 = be_ref[0, 0]
    gamma0 = ga_ref[0, 0]

    pre1 = (ur * w1u_ref[...] + sr * w1s_ref[...]) + b1_ref[...]
    h1 = jax.nn.sigmoid(pre1)
    h2 = jax.nn.sigmoid(
        jnp.dot(h1, w2_ref[...], preferred_element_type=jnp.float32)
        + b2_ref[...])
    apre = jnp.sum(h2 * w3_ref[...], axis=1, keepdims=True) + b3_ref[0, 0]
    alphas = jax.nn.sigmoid(apre) * alpha0

    u1 = ur + (alphas - beta0 * ur) * DT
    s1 = sr + (beta0 * ur - gamma0 * sr) * DT
    uv = u1 - ur
    sv = s1 - sr

    sqa = sa * sa + ua * ua
    sqr = sr * sr + ur * ur
    dotv = jnp.dot(p8r_ref[...], p8c_ref[...],
                   preferred_element_type=jnp.float32)
    d2 = (sqr + sqa) - 2.0 * dotv

    gmin = jnp.min(d2, axis=1, keepdims=True)

    kf = jnp.float32(KSEL)

    cm = jnp.minimum(d2[:, :4096], d2[:, 4096:])
    cm = jnp.minimum(cm[:, :2048], cm[:, 2048:])
    cm = jnp.minimum(cm[:, :1024], cm[:, 1024:])
    cm = jnp.minimum(cm[:, :512], cm[:, 512:])
    cm = jnp.minimum(cm[:, :256], cm[:, 256:])
    cm = jnp.minimum(cm[:, :128], cm[:, 128:])
    cm = jnp.minimum(cm[:, :64], cm[:, 64:])
    for _ in range(KSEL - 1):
        vm = jnp.min(cm, axis=1, keepdims=True)
        cm = jnp.where(cm == vm, jnp.inf, cm)
    ub = jnp.min(cm, axis=1, keepdims=True)

    cle0 = jnp.sum((d2 <= gmin).astype(jnp.float32), axis=1, keepdims=True)
    done0 = cle0 >= kf
    lo = gmin
    hi = ub
    clo = cle0
    for _ in range(8):
        mid = 0.5 * (lo + hi)
        c = jnp.sum((d2 <= mid).astype(jnp.float32), axis=1, keepdims=True)
        ge = c >= kf
        hi = jnp.where(jnp.logical_not(done0) & ge, mid, hi)
        movelo = jnp.logical_not(done0) & jnp.logical_not(ge)
        lo = jnp.where(movelo, mid, lo)
        clo = jnp.where(movelo, c, clo)

    stat_ref[:, 0:1] = lo
    stat_ref[:, 1:2] = jnp.where(done0, gmin, lo)
    stat_ref[:, 2:3] = jnp.where(done0, 0.0, clo)
    stat_ref[:, 3:4] = clo
    stat_ref[:, 4:5] = done0.astype(jnp.float32)

    def walk_cond(nleft):
        return nleft > 0.0

    def walk_body(nleft):
        tcur = stat_ref[:, 0:1]
        clo_ = stat_ref[:, 3:4]
        done = stat_ref[:, 4:5] > 0.0
        vnext = jnp.min(jnp.where(d2 > tcur, d2, jnp.inf),
                        axis=1, keepdims=True)
        ceq = jnp.sum((d2 == vnext).astype(jnp.float32),
                      axis=1, keepdims=True)
        cnext = clo_ + ceq
        cross = jnp.logical_not(done) & (cnext >= kf)
        stat_ref[:, 1:2] = jnp.where(cross, vnext, stat_ref[:, 1:2])
        stat_ref[:, 2:3] = jnp.where(cross, clo_, stat_ref[:, 2:3])
        nd = done | cross
        stat_ref[:, 4:5] = nd.astype(jnp.float32)
        stat_ref[:, 0:1] = jnp.where(nd, tcur, vnext)
        stat_ref[:, 3:4] = jnp.where(nd, clo_, cnext)
        return jnp.sum(1.0 - stat_ref[:, 4:5])

    jax.lax.while_loop(walk_cond, walk_body,
                       jnp.sum(1.0 - stat_ref[:, 4:5]))

    t = stat_ref[:, 1:2]
    clt = stat_ref[:, 2:3]

    tie = d2 == t
    ntie = jnp.sum(tie.astype(jnp.float32), axis=1, keepdims=True)
    excess0 = ntie - (kf - clt)
    keep_ref[...] = tie.astype(jnp.float32)
    stat_ref[:, 5:6] = excess0
    idx = jax.lax.broadcasted_iota(jnp.int32, (ROWS, N), 1)

    def trim_cond(nleft):
        return nleft > 0.0

    def trim_body(nleft):
        excess = stat_ref[:, 5:6]
        km = keep_ref[...] > 0.0
        jmax = jnp.max(jnp.where(km, idx, -1), axis=1, keepdims=True)
        upd = excess > 0.0
        rm = upd & (idx == jmax)
        keep_ref[...] = jnp.where(rm, 0.0, keep_ref[...])
        exnew = jnp.where(upd, excess - 1.0, excess)
        stat_ref[:, 5:6] = exnew
        return jnp.max(exnew)

    jax.lax.while_loop(trim_cond, trim_body, jnp.max(excess0))

    jmin0 = jnp.min(jnp.where(d2 == gmin, idx, N), axis=1, keepdims=True)
    first = (d2 == gmin) & (idx == jmin0)
    included = ((d2 < t) | (keep_ref[...] > 0.0)) & jnp.logical_not(first)

    unv = ua - ur
    snv = sa - sr
    n2 = unv * unv + snv * snv
    v2 = uv * uv + sv * sv
    num = unv * uv + snv * sv
    zero_den = (n2 == 0.0) | (v2 == 0.0)
    cosine = jnp.where(zero_den, 1.0, num / jnp.sqrt(n2 * v2))
    cosmax = jnp.max(jnp.where(included, cosine, -3.0), axis=1, keepdims=True)

    cost_ref[...] = 1.0 - cosmax
    u1_ref[...] = u1
    s1_ref[...] = s1
    al_ref[...] = alphas


def kernel(u0, s0, alpha0, beta0, gamma0, W1, b1, W2, b2, W3, b3):
    f32 = jnp.float32
    u_col = u0.reshape(N, 1)
    s_col = s0.reshape(N, 1)
    u_row = u0.reshape(1, N)
    s_row = s0.reshape(1, N)
    pts8 = jnp.pad(jnp.stack([s0, u0], axis=1), ((0, 0), (0, 6)))
    pts8t = pts8.T
    hp = HPAD - HID
    w1u = jnp.pad(W1[0:1, :], ((0, 0), (0, hp)))
    w1s = jnp.pad(W1[1:2, :], ((0, 0), (0, hp)))
    b1p = jnp.pad(b1.reshape(1, HID), ((0, 0), (0, hp)))
    w2p = jnp.pad(W2, ((0, hp), (0, hp)))
    b2p = jnp.pad(b2.reshape(1, HID), ((0, 0), (0, hp)))
    w3p = jnp.pad(W3.reshape(1, HID), ((0, 0), (0, hp)))
    b3p = b3.reshape(1, 1)
    a0 = alpha0.reshape(1, 1).astype(f32)
    be = beta0.reshape(1, 1).astype(f32)
    ga = gamma0.reshape(1, 1).astype(f32)

    grid = N // ROWS
    row_spec = pl.BlockSpec((ROWS, 1), lambda i: (i, 0))
    full_spec = pl.BlockSpec((1, N), lambda i: (0, 0))

    def fixed(shape):
        return pl.BlockSpec(shape, lambda i: (0, 0))

    cost2, u12, s12, al2 = pl.pallas_call(
        _body,
        grid=(grid,),
        in_specs=[row_spec, row_spec, full_spec, full_spec,
                  pl.BlockSpec((ROWS, 8), lambda i: (i, 0)),
                  pl.BlockSpec((8, N), lambda i: (0, 0)),
                  fixed((1, HPAD)), fixed((1, HPAD)), fixed((1, HPAD)),
                  fixed((HPAD, HPAD)), fixed((1, HPAD)), fixed((1, HPAD)),
                  fixed((1, 1)), fixed((1, 1)), fixed((1, 1)), fixed((1, 1))],
        out_specs=[row_spec, row_spec, row_spec, row_spec],
        out_shape=[jax.ShapeDtypeStruct((N, 1), f32) for _ in range(4)],
        scratch_shapes=[pltpu.VMEM((ROWS, N), f32),
                        pltpu.VMEM((ROWS, 128), f32)],
    )(u_col, s_col, u_row, s_row, pts8, pts8t,
      w1u, w1s, b1p, w2p, b2p, w3p, b3p, a0, be, ga)

    cost = cost2.reshape(N)
    u1 = u12.reshape(N)
    s1 = s12.reshape(N)
    alphas = al2.reshape(N)
    beta = jnp.broadcast_to(beta0, u0.shape)
    gamma = jnp.broadcast_to(gamma0, u0.shape)
    return (cost, u1, s1, alphas, beta, gamma)


# speedup vs baseline: 2.4673x; 1.0786x over previous
import jax
import jax.numpy as jnp
from jax.experimental import pallas as pl
from jax.experimental.pallas import tpu as pltpu

N = 8192
KSEL = 30
ROWS = 256
HID = 100
HPAD = 128
DT = 0.5


def _body(ur_ref, sr_ref, ua_ref, sa_ref, p8r_ref, p8c_ref,
          w1u_ref, w1s_ref, b1_ref, w2_ref, b2_ref, w3_ref, b3_ref,
          a0_ref, be_ref, ga_ref,
          cost_ref, u1_ref, s1_ref, al_ref,
          keep_ref, stat_ref):
    ur = ur_ref[...]
    sr = sr_ref[...]
    ua = ua_ref[...]
    sa = sa_ref[...]
    alpha0 = a0_ref[0, 0]
    beta0 = be_ref[0, 0]
    gamma0 = ga_ref[0, 0]

    pre1 = (ur * w1u_ref[...] + sr * w1s_ref[...]) + b1_ref[...]
    h1 = jax.nn.sigmoid(pre1)
    h2 = jax.nn.sigmoid(
        jnp.dot(h1, w2_ref[...], preferred_element_type=jnp.float32)
        + b2_ref[...])
    apre = jnp.sum(h2 * w3_ref[...], axis=1, keepdims=True) + b3_ref[0, 0]
    alphas = jax.nn.sigmoid(apre) * alpha0

    u1 = ur + (alphas - beta0 * ur) * DT
    s1 = sr + (beta0 * ur - gamma0 * sr) * DT
    uv = u1 - ur
    sv = s1 - sr

    sqa = sa * sa + ua * ua
    sqr = sr * sr + ur * ur
    dotv = jnp.dot(p8r_ref[...], p8c_ref[...],
                   preferred_element_type=jnp.float32)
    d2 = (sqr + sqa) - 2.0 * dotv

    gmin = jnp.min(d2, axis=1, keepdims=True)

    kf = jnp.float32(KSEL)

    cm = jnp.minimum(d2[:, :4096], d2[:, 4096:])
    cm = jnp.minimum(cm[:, :2048], cm[:, 2048:])
    cm = jnp.minimum(cm[:, :1024], cm[:, 1024:])
    cm = jnp.minimum(cm[:, :512], cm[:, 512:])
    cm = jnp.minimum(cm[:, :256], cm[:, 256:])
    cm = jnp.minimum(cm[:, :128], cm[:, 128:])
    cm = jnp.minimum(cm[:, :64], cm[:, 64:])
    for _ in range(KSEL - 1):
        vm = jnp.min(cm, axis=1, keepdims=True)
        cm = jnp.where(cm == vm, jnp.inf, cm)
    ub = jnp.min(cm, axis=1, keepdims=True)

    cle0 = jnp.sum((d2 <= gmin).astype(jnp.float32), axis=1, keepdims=True)
    done0 = cle0 >= kf
    lo = gmin
    hi = ub
    clo = cle0
    for _ in range(8):
        mid = 0.5 * (lo + hi)
        c = jnp.sum((d2 <= mid).astype(jnp.float32), axis=1, keepdims=True)
        ge = c >= kf
        hi = jnp.where(jnp.logical_not(done0) & ge, mid, hi)
        movelo = jnp.logical_not(done0) & jnp.logical_not(ge)
        lo = jnp.where(movelo, mid, lo)
        clo = jnp.where(movelo, c, clo)

    stat_ref[:, 0:1] = lo
    stat_ref[:, 1:2] = jnp.where(done0, gmin, lo)
    stat_ref[:, 2:3] = jnp.where(done0, 0.0, clo)
    stat_ref[:, 3:4] = clo
    stat_ref[:, 4:5] = done0.astype(jnp.float32)

    def walk_cond(nleft):
        return nleft > 0.0

    def walk_body(nleft):
        tcur = stat_ref[:, 0:1]
        clo_ = stat_ref[:, 3:4]
        done = stat_ref[:, 4:5] > 0.0
        vnext = jnp.min(jnp.where(d2 > tcur, d2, jnp.inf),
                        axis=1, keepdims=True)
        ceq = jnp.sum((d2 == vnext).astype(jnp.float32),
                      axis=1, keepdims=True)
        cnext = clo_ + ceq
        cross = jnp.logical_not(done) & (cnext >= kf)
        stat_ref[:, 1:2] = jnp.where(cross, vnext, stat_ref[:, 1:2])
        stat_ref[:, 2:3] = jnp.where(cross, clo_, stat_ref[:, 2:3])
        nd = done | cross
        stat_ref[:, 4:5] = nd.astype(jnp.float32)
        stat_ref[:, 0:1] = jnp.where(nd, tcur, vnext)
        stat_ref[:, 3:4] = jnp.where(nd, clo_, cnext)
        return jnp.sum(1.0 - stat_ref[:, 4:5])

    jax.lax.while_loop(walk_cond, walk_body,
                       jnp.sum(1.0 - stat_ref[:, 4:5]))

    t = stat_ref[:, 1:2]
    clt = stat_ref[:, 2:3]

    tie = d2 == t
    ntie = jnp.sum(tie.astype(jnp.float32), axis=1, keepdims=True)
    excess0 = ntie - (kf - clt)
    keep_ref[...] = tie.astype(jnp.float32)
    stat_ref[:, 5:6] = excess0
    idx = jax.lax.broadcasted_iota(jnp.int32, (ROWS, N), 1)

    def trim_cond(nleft):
        return nleft > 0.0

    def trim_body(nleft):
        excess = stat_ref[:, 5:6]
        km = keep_ref[...] > 0.0
        jmax = jnp.max(jnp.where(km, idx, -1), axis=1, keepdims=True)
        upd = excess > 0.0
        rm = upd & (idx == jmax)
        keep_ref[...] = jnp.where(rm, 0.0, keep_ref[...])
        exnew = jnp.where(upd, excess - 1.0, excess)
        stat_ref[:, 5:6] = exnew
        return jnp.max(exnew)

    jax.lax.while_loop(trim_cond, trim_body, jnp.max(excess0))

    jmin0 = jnp.min(jnp.where(d2 == gmin, idx, N), axis=1, keepdims=True)
    first = (d2 == gmin) & (idx == jmin0)
    included = ((d2 < t) | (keep_ref[...] > 0.0)) & jnp.logical_not(first)

    unv = ua - ur
    snv = sa - sr
    n2 = unv * unv + snv * snv
    v2 = uv * uv + sv * sv
    num = unv * uv + snv * sv
    q = jnp.where(n2 == 0.0, jnp.inf, num / jnp.sqrt(n2))
    qmax = jnp.max(jnp.where(included, q, -jnp.inf), axis=1, keepdims=True)
    cosmax = jnp.where((v2 == 0.0) | (qmax == jnp.inf),
                       1.0, qmax / jnp.sqrt(v2))
    cost_ref[...] = 1.0 - cosmax
    u1_ref[...] = u1
    s1_ref[...] = s1
    al_ref[...] = alphas


def kernel(u0, s0, alpha0, beta0, gamma0, W1, b1, W2, b2, W3, b3):
    f32 = jnp.float32
    u_col = u0.reshape(N, 1)
    s_col = s0.reshape(N, 1)
    u_row = u0.reshape(1, N)
    s_row = s0.reshape(1, N)
    pts8 = jnp.pad(jnp.stack([s0, u0], axis=1), ((0, 0), (0, 6)))
    pts8t = pts8.T
    hp = HPAD - HID
    w1u = jnp.pad(W1[0:1, :], ((0, 0), (0, hp)))
    w1s = jnp.pad(W1[1:2, :], ((0, 0), (0, hp)))
    b1p = jnp.pad(b1.reshape(1, HID), ((0, 0), (0, hp)))
    w2p = jnp.pad(W2, ((0, hp), (0, hp)))
    b2p = jnp.pad(b2.reshape(1, HID), ((0, 0), (0, hp)))
    w3p = jnp.pad(W3.reshape(1, HID), ((0, 0), (0, hp)))
    b3p = b3.reshape(1, 1)
    a0 = alpha0.reshape(1, 1).astype(f32)
    be = beta0.reshape(1, 1).astype(f32)
    ga = gamma0.reshape(1, 1).astype(f32)

    grid = N // ROWS
    row_spec = pl.BlockSpec((ROWS, 1), lambda i: (i, 0))
    full_spec = pl.BlockSpec((1, N), lambda i: (0, 0))

    def fixed(shape):
        return pl.BlockSpec(shape, lambda i: (0, 0))

    cost2, u12, s12, al2 = pl.pallas_call(
        _body,
        grid=(grid,),
        in_specs=[row_spec, row_spec, full_spec, full_spec,
                  pl.BlockSpec((ROWS, 8), lambda i: (i, 0)),
                  pl.BlockSpec((8, N), lambda i: (0, 0)),
                  fixed((1, HPAD)), fixed((1, HPAD)), fixed((1, HPAD)),
                  fixed((HPAD, HPAD)), fixed((1, HPAD)), fixed((1, HPAD)),
                  fixed((1, 1)), fixed((1, 1)), fixed((1, 1)), fixed((1, 1))],
        out_specs=[row_spec, row_spec, row_spec, row_spec],
        out_shape=[jax.ShapeDtypeStruct((N, 1), f32) for _ in range(4)],
        scratch_shapes=[pltpu.VMEM((ROWS, N), f32),
                        pltpu.VMEM((ROWS, 128), f32)],
        compiler_params=pltpu.CompilerParams(
            dimension_semantics=("parallel",)),
    )(u_col, s_col, u_row, s_row, pts8, pts8t,
      w1u, w1s, b1p, w2p, b2p, w3p, b3p, a0, be, ga)

    cost = cost2.reshape(N)
    u1 = u12.reshape(N)
    s1 = s12.reshape(N)
    alphas = al2.reshape(N)
    beta = jnp.broadcast_to(beta0, u0.shape)
    gamma = jnp.broadcast_to(gamma0, u0.shape)
    return (cost, u1, s1, alphas, beta, gamma)


# speedup vs baseline: 2.5625x; 1.0386x over previous
import jax
import jax.numpy as jnp
from jax.experimental import pallas as pl
from jax.experimental.pallas import tpu as pltpu

N = 8192
KSEL = 30
ROWS = 256
HID = 100
HPAD = 128
DT = 0.5


def _body(ur_ref, sr_ref, ua_ref, sa_ref, p8r_ref, p8c_ref,
          w1u_ref, w1s_ref, b1_ref, w2_ref, b2_ref, w3_ref, b3_ref,
          a0_ref, be_ref, ga_ref,
          cost_ref, u1_ref, s1_ref, al_ref,
          keep_ref, stat_ref):
    ur = ur_ref[...]
    sr = sr_ref[...]
    ua = ua_ref[...]
    sa = sa_ref[...]
    alpha0 = a0_ref[0, 0]
    beta0 = be_ref[0, 0]
    gamma0 = ga_ref[0, 0]

    pre1 = (ur * w1u_ref[...] + sr * w1s_ref[...]) + b1_ref[...]
    h1 = jax.nn.sigmoid(pre1)
    h2 = jax.nn.sigmoid(
        jnp.dot(h1, w2_ref[...], preferred_element_type=jnp.float32)
        + b2_ref[...])
    apre = jnp.sum(h2 * w3_ref[...], axis=1, keepdims=True) + b3_ref[0, 0]
    alphas = jax.nn.sigmoid(apre) * alpha0

    u1 = ur + (alphas - beta0 * ur) * DT
    s1 = sr + (beta0 * ur - gamma0 * sr) * DT
    uv = u1 - ur
    sv = s1 - sr

    sqa = sa * sa + ua * ua
    sqr = sr * sr + ur * ur
    dotv = jnp.dot(p8r_ref[...], p8c_ref[...],
                   preferred_element_type=jnp.float32)
    d2 = (sqr + sqa) - 2.0 * dotv

    gmin = jnp.min(d2, axis=1, keepdims=True)

    kf = jnp.float32(KSEL)

    cm = jnp.minimum(d2[:, :4096], d2[:, 4096:])
    cm = jnp.minimum(cm[:, :2048], cm[:, 2048:])
    cm = jnp.minimum(cm[:, :1024], cm[:, 1024:])
    cm = jnp.minimum(cm[:, :512], cm[:, 512:])
    cm = jnp.minimum(cm[:, :256], cm[:, 256:])
    cm = jnp.minimum(cm[:, :128], cm[:, 128:])
    cm = jnp.minimum(cm[:, :64], cm[:, 64:])
    for _ in range(KSEL - 1):
        vm = jnp.min(cm, axis=1, keepdims=True)
        cm = jnp.where(cm == vm, jnp.inf, cm)
    ub = jnp.min(cm, axis=1, keepdims=True)

    cle0 = jnp.sum((d2 <= gmin).astype(jnp.float32), axis=1, keepdims=True)
    done0 = cle0 >= kf
    lo = gmin
    hi = ub
    clo = cle0
    for _ in range(8):
        mid = 0.5 * (lo + hi)
        c = jnp.sum((d2 <= mid).astype(jnp.float32), axis=1, keepdims=True)
        ge = c >= kf
        hi = jnp.where(jnp.logical_not(done0) & ge, mid, hi)
        movelo = jnp.logical_not(done0) & jnp.logical_not(ge)
        lo = jnp.where(movelo, mid, lo)
        clo = jnp.where(movelo, c, clo)

    stat_ref[:, 0:1] = lo
    stat_ref[:, 1:2] = jnp.where(done0, gmin, lo)
    stat_ref[:, 2:3] = jnp.where(done0, 0.0, clo)
    stat_ref[:, 3:4] = clo
    stat_ref[:, 4:5] = done0.astype(jnp.float32)

    def walk_cond(nleft):
        return nleft > 0.0

    def walk_body(nleft):
        tcur = stat_ref[:, 0:1]
        clo_ = stat_ref[:, 3:4]
        done = stat_ref[:, 4:5] > 0.0
        vnext = jnp.min(jnp.where(d2 > tcur, d2, jnp.inf),
                        axis=1, keepdims=True)
        ceq = jnp.sum((d2 == vnext).astype(jnp.float32),
                      axis=1, keepdims=True)
        cnext = clo_ + ceq
        cross = jnp.logical_not(done) & (cnext >= kf)
        stat_ref[:, 1:2] = jnp.where(cross, vnext, stat_ref[:, 1:2])
        stat_ref[:, 2:3] = jnp.where(cross, clo_, stat_ref[:, 2:3])
        nd = done | cross
        stat_ref[:, 4:5] = nd.astype(jnp.float32)
        stat_ref[:, 0:1] = jnp.where(nd, tcur, vnext)
        stat_ref[:, 3:4] = jnp.where(nd, clo_, cnext)
        return jnp.sum(1.0 - stat_ref[:, 4:5])

    jax.lax.while_loop(walk_cond, walk_body,
                       jnp.sum(1.0 - stat_ref[:, 4:5]))

    t = stat_ref[:, 1:2]
    clt = stat_ref[:, 2:3]

    tie = d2 == t
    ntie = jnp.sum(tie.astype(jnp.float32), axis=1, keepdims=True)
    excess0 = ntie - (kf - clt)
    keep_ref[...] = tie.astype(jnp.float32)
    stat_ref[:, 5:6] = excess0
    idx = jax.lax.broadcasted_iota(jnp.int32, (ROWS, N), 1)

    def trim_cond(nleft):
        return nleft > 0.0

    def trim_body(nleft):
        excess = stat_ref[:, 5:6]
        km = keep_ref[...] > 0.0
        jmax = jnp.max(jnp.where(km, idx, -1), axis=1, keepdims=True)
        upd = excess > 0.0
        rm = upd & (idx == jmax)
        keep_ref[...] = jnp.where(rm, 0.0, keep_ref[...])
        exnew = jnp.where(upd, excess - 1.0, excess)
        stat_ref[:, 5:6] = exnew
        return jnp.max(exnew)

    jax.lax.while_loop(trim_cond, trim_body, jnp.max(excess0))

    jmin0 = jnp.min(jnp.where(d2 == gmin, idx, N), axis=1, keepdims=True)
    first = (d2 == gmin) & (idx == jmin0)
    included = ((d2 < t) | (keep_ref[...] > 0.0)) & jnp.logical_not(first)

    unv = ua - ur
    snv = sa - sr
    n2 = unv * unv + snv * snv
    v2 = uv * uv + sv * sv
    num = unv * uv + snv * sv
    g = jnp.where(n2 == 0.0, jnp.inf, (num * jnp.abs(num)) / n2)
    gmax = jnp.max(jnp.where(included, g, -jnp.inf), axis=1, keepdims=True)
    qmax = jnp.sign(gmax) * jnp.sqrt(jnp.abs(gmax))
    cosmax = jnp.where((v2 == 0.0) | (gmax == jnp.inf),
                       1.0, qmax / jnp.sqrt(v2))
    cost_ref[...] = 1.0 - cosmax
    u1_ref[...] = u1
    s1_ref[...] = s1
    al_ref[...] = alphas


def kernel(u0, s0, alpha0, beta0, gamma0, W1, b1, W2, b2, W3, b3):
    f32 = jnp.float32
    u_col = u0.reshape(N, 1)
    s_col = s0.reshape(N, 1)
    u_row = u0.reshape(1, N)
    s_row = s0.reshape(1, N)
    pts8 = jnp.pad(jnp.stack([s0, u0], axis=1), ((0, 0), (0, 6)))
    pts8t = pts8.T
    hp = HPAD - HID
    w1u = jnp.pad(W1[0:1, :], ((0, 0), (0, hp)))
    w1s = jnp.pad(W1[1:2, :], ((0, 0), (0, hp)))
    b1p = jnp.pad(b1.reshape(1, HID), ((0, 0), (0, hp)))
    w2p = jnp.pad(W2, ((0, hp), (0, hp)))
    b2p = jnp.pad(b2.reshape(1, HID), ((0, 0), (0, hp)))
    w3p = jnp.pad(W3.reshape(1, HID), ((0, 0), (0, hp)))
    b3p = b3.reshape(1, 1)
    a0 = alpha0.reshape(1, 1).astype(f32)
    be = beta0.reshape(1, 1).astype(f32)
    ga = gamma0.reshape(1, 1).astype(f32)

    grid = N // ROWS
    row_spec = pl.BlockSpec((ROWS, 1), lambda i: (i, 0))
    full_spec = pl.BlockSpec((1, N), lambda i: (0, 0))

    def fixed(shape):
        return pl.BlockSpec(shape, lambda i: (0, 0))

    cost2, u12, s12, al2 = pl.pallas_call(
        _body,
        grid=(grid,),
        in_specs=[row_spec, row_spec, full_spec, full_spec,
                  pl.BlockSpec((ROWS, 8), lambda i: (i, 0)),
                  pl.BlockSpec((8, N), lambda i: (0, 0)),
                  fixed((1, HPAD)), fixed((1, HPAD)), fixed((1, HPAD)),
                  fixed((HPAD, HPAD)), fixed((1, HPAD)), fixed((1, HPAD)),
                  fixed((1, 1)), fixed((1, 1)), fixed((1, 1)), fixed((1, 1))],
        out_specs=[row_spec, row_spec, row_spec, row_spec],
        out_shape=[jax.ShapeDtypeStruct((N, 1), f32) for _ in range(4)],
        scratch_shapes=[pltpu.VMEM((ROWS, N), f32),
                        pltpu.VMEM((ROWS, 128), f32)],
        compiler_params=pltpu.CompilerParams(
            dimension_semantics=("parallel",)),
    )(u_col, s_col, u_row, s_row, pts8, pts8t,
      w1u, w1s, b1p, w2p, b2p, w3p, b3p, a0, be, ga)

    cost = cost2.reshape(N)
    u1 = u12.reshape(N)
    s1 = s12.reshape(N)
    alphas = al2.reshape(N)
    beta = jnp.broadcast_to(beta0, u0.shape)
    gamma = jnp.broadcast_to(gamma0, u0.shape)
    return (cost, u1, s1, alphas, beta, gamma)


# speedup vs baseline: 2.5744x; 1.0046x over previous
import jax
import jax.numpy as jnp
from jax.experimental import pallas as pl
from jax.experimental.pallas import tpu as pltpu

N = 8192
KSEL = 30
ROWS = 256
HID = 100
HPAD = 128
DT = 0.5


def _sig(x):
    return 1.0 / (1.0 + jnp.exp(-x))


def _body(ur_ref, sr_ref, ua_ref, sa_ref, p8r_ref, p8c_ref, m8r_ref,
          w18_ref, b1_ref, w2_ref, b2_ref, w3_ref, b3_ref,
          a0_ref, be_ref, ga_ref,
          cost_ref, u1_ref, s1_ref, al_ref,
          keep_ref, stat_ref):
    ur = ur_ref[...]
    sr = sr_ref[...]
    ua = ua_ref[...]
    sa = sa_ref[...]
    alpha0 = a0_ref[0, 0]
    beta0 = be_ref[0, 0]
    gamma0 = ga_ref[0, 0]

    w18 = w18_ref[...]
    pre1 = (ur * w18[0:1, :] + sr * w18[1:2, :]) + b1_ref[...]
    h1 = _sig(pre1)
    h2 = _sig(
        jnp.dot(h1, w2_ref[...], preferred_element_type=jnp.float32)
        + b2_ref[...])
    apre = jnp.sum(h2 * w3_ref[...], axis=1, keepdims=True) + b3_ref[0, 0]
    alphas = _sig(apre) * alpha0

    u1 = ur + (alphas - beta0 * ur) * DT
    s1 = sr + (beta0 * ur - gamma0 * sr) * DT
    uv = u1 - ur
    sv = s1 - sr

    sqa = sa * sa + ua * ua
    sqr = sr * sr + ur * ur
    dotv = jnp.dot(p8r_ref[...], p8c_ref[...],
                   preferred_element_type=jnp.float32)
    d2 = (sqr + sqa) - 2.0 * dotv

    gmin = jnp.min(d2, axis=1, keepdims=True)

    kf = jnp.float32(KSEL)

    cm = jnp.minimum(d2[:, :4096], d2[:, 4096:])
    cm = jnp.minimum(cm[:, :2048], cm[:, 2048:])
    cm = jnp.minimum(cm[:, :1024], cm[:, 1024:])
    cm = jnp.minimum(cm[:, :512], cm[:, 512:])
    cm = jnp.minimum(cm[:, :256], cm[:, 256:])
    cm = jnp.minimum(cm[:, :128], cm[:, 128:])
    cm = jnp.minimum(cm[:, :64], cm[:, 64:])
    for _ in range(KSEL - 1):
        vm = jnp.min(cm, axis=1, keepdims=True)
        cm = jnp.where(cm == vm, jnp.inf, cm)
    ub = jnp.min(cm, axis=1, keepdims=True)

    lo = gmin - (jnp.abs(gmin) * 1e-5 + 1e-30)
    hi = ub
    clo = jnp.zeros((ROWS, 1), jnp.float32)
    for _ in range(7):
        mid = 0.5 * (lo + hi)
        c = jnp.sum((d2 <= mid).astype(jnp.float32), axis=1, keepdims=True)
        ge = c >= kf
        hi = jnp.where(ge, mid, hi)
        lo = jnp.where(ge, lo, mid)
        clo = jnp.where(ge, clo, c)

    stat_ref[:, 0:1] = lo
    stat_ref[:, 1:2] = lo
    stat_ref[:, 2:3] = clo
    stat_ref[:, 3:4] = clo
    stat_ref[:, 4:5] = jnp.zeros((ROWS, 1), jnp.float32)

    def walk_cond(nleft):
        return nleft > 0.0

    def walk_body(nleft):
        tcur = stat_ref[:, 0:1]
        clo_ = stat_ref[:, 3:4]
        done = stat_ref[:, 4:5] > 0.0
        vnext = jnp.min(jnp.where(d2 > tcur, d2, jnp.inf),
                        axis=1, keepdims=True)
        ceq = jnp.sum((d2 == vnext).astype(jnp.float32),
                      axis=1, keepdims=True)
        cnext = clo_ + ceq
        cross = jnp.logical_not(done) & (cnext >= kf)
        stat_ref[:, 1:2] = jnp.where(cross, vnext, stat_ref[:, 1:2])
        stat_ref[:, 2:3] = jnp.where(cross, clo_, stat_ref[:, 2:3])
        nd = done | cross
        stat_ref[:, 4:5] = nd.astype(jnp.float32)
        stat_ref[:, 0:1] = jnp.where(nd, tcur, vnext)
        stat_ref[:, 3:4] = jnp.where(nd, clo_, cnext)
        return jnp.sum(1.0 - stat_ref[:, 4:5])

    jax.lax.while_loop(walk_cond, walk_body,
                       jnp.sum(1.0 - stat_ref[:, 4:5]))

    t = stat_ref[:, 1:2]
    clt = stat_ref[:, 2:3]

    tie = d2 == t
    ntie = jnp.sum(tie.astype(jnp.float32), axis=1, keepdims=True)
    excess0 = ntie - (kf - clt)
    keep_ref[...] = tie.astype(jnp.float32)
    stat_ref[:, 5:6] = excess0
    idx = jax.lax.broadcasted_iota(jnp.int32, (ROWS, N), 1)

    def trim_cond(nleft):
        return nleft > 0.0

    def trim_body(nleft):
        excess = stat_ref[:, 5:6]
        km = keep_ref[...] > 0.0
        jmax = jnp.max(jnp.where(km, idx, -1), axis=1, keepdims=True)
        upd = excess > 0.0
        rm = upd & (idx == jmax)
        keep_ref[...] = jnp.where(rm, 0.0, keep_ref[...])
        exnew = jnp.where(upd, excess - 1.0, excess)
        stat_ref[:, 5:6] = exnew
        return jnp.max(exnew)

    jax.lax.while_loop(trim_cond, trim_body, jnp.max(excess0))

    jmin0 = jnp.min(jnp.where(d2 == gmin, idx, N), axis=1, keepdims=True)
    first = (d2 == gmin) & (idx == jmin0)
    included = ((d2 < t) | (keep_ref[...] > 0.0)) & jnp.logical_not(first)

    unv = ua - ur
    snv = sa - sr
    n2 = unv * unv + snv * snv
    v2 = uv * uv + sv * sv
    num = unv * uv + snv * sv
    g = jnp.where(n2 == 0.0, jnp.inf, (num * jnp.abs(num)) / n2)
    gmax = jnp.max(jnp.where(included, g, -jnp.inf), axis=1, keepdims=True)
    qmax = jnp.sign(gmax) * jnp.sqrt(jnp.abs(gmax))
    cosmax = jnp.where((v2 == 0.0) | (gmax == jnp.inf),
                       1.0, qmax / jnp.sqrt(v2))
    cost_ref[...] = 1.0 - cosmax
    u1_ref[...] = u1
    s1_ref[...] = s1
    al_ref[...] = alphas


def kernel(u0, s0, alpha0, beta0, gamma0, W1, b1, W2, b2, W3, b3):
    f32 = jnp.float32
    u_col = u0.reshape(N, 1)
    s_col = s0.reshape(N, 1)
    u_row = u0.reshape(1, N)
    s_row = s0.reshape(1, N)
    pts8 = jnp.pad(jnp.stack([s0, u0], axis=1), ((0, 0), (0, 6)))
    pts8t = pts8.T
    inp8 = jnp.pad(jnp.stack([u0, s0], axis=1), ((0, 0), (0, 6)))
    hp = HPAD - HID
    w18 = jnp.pad(W1, ((0, 6), (0, hp)))
    b1p = jnp.pad(b1.reshape(1, HID), ((0, 0), (0, hp)))
    w2p = jnp.pad(W2, ((0, hp), (0, hp)))
    b2p = jnp.pad(b2.reshape(1, HID), ((0, 0), (0, hp)))
    w3p = jnp.pad(W3.reshape(1, HID), ((0, 0), (0, hp)))
    b3p = b3.reshape(1, 1)
    a0 = alpha0.reshape(1, 1).astype(f32)
    be = beta0.reshape(1, 1).astype(f32)
    ga = gamma0.reshape(1, 1).astype(f32)

    grid = N // ROWS
    row_spec = pl.BlockSpec((ROWS, 1), lambda i: (i, 0))
    full_spec = pl.BlockSpec((1, N), lambda i: (0, 0))

    def fixed(shape):
        return pl.BlockSpec(shape, lambda i: (0, 0))

    cost2, u12, s12, al2 = pl.pallas_call(
        _body,
        grid=(grid,),
        in_specs=[row_spec, row_spec, full_spec, full_spec,
                  pl.BlockSpec((ROWS, 8), lambda i: (i, 0)),
                  pl.BlockSpec((8, N), lambda i: (0, 0)),
                  pl.BlockSpec((ROWS, 8), lambda i: (i, 0)),
                  fixed((8, HPAD)), fixed((1, HPAD)),
                  fixed((HPAD, HPAD)), fixed((1, HPAD)), fixed((1, HPAD)),
                  fixed((1, 1)), fixed((1, 1)), fixed((1, 1)), fixed((1, 1))],
        out_specs=[row_spec, row_spec, row_spec, row_spec],
        out_shape=[jax.ShapeDtypeStruct((N, 1), f32) for _ in range(4)],
        scratch_shapes=[pltpu.VMEM((ROWS, N), f32),
                        pltpu.VMEM((ROWS, 128), f32)],
        compiler_params=pltpu.CompilerParams(
            dimension_semantics=("parallel",)),
    )(u_col, s_col, u_row, s_row, pts8, pts8t, inp8,
      w18, b1p, w2p, b2p, w3p, b3p, a0, be, ga)

    cost = cost2.reshape(N)
    u1 = u12.reshape(N)
    s1 = s12.reshape(N)
    alphas = al2.reshape(N)
    beta = jnp.broadcast_to(beta0, u0.shape)
    gamma = jnp.broadcast_to(gamma0, u0.shape)
    return (cost, u1, s1, alphas, beta, gamma)
